# Initial kernel scaffold; baseline (speedup 1.0000x reference)
import jax
import jax.numpy as jnp
from jax.experimental import pallas as pl


def kernel(x, k, W_enc, b_enc, W_dec):
    raise NotImplementedError("write your pallas kernel here")


# speedup vs baseline: 12.5685x; 12.5685x over previous
import functools

import jax
import jax.numpy as jnp
from jax import lax
from jax.experimental import pallas as pl
from jax.experimental.pallas import tpu as pltpu

N_TOK = 8192
D_IN = 1024
D_HID = 4096
TOPK_WIDTH = 64
BT = 256


def _fused_body(k_ref, x_ref, we_ref, be_ref, wd_ref, recon_ref, h_ref):
    h_pre = lax.dot_general(
        x_ref[...], we_ref[...],
        dimension_numbers=(((1,), (1,)), ((), ())),
        preferred_element_type=jnp.float32,
    )
    h = jnp.maximum(h_pre + be_ref[...], 0.0)

    kk = jnp.minimum(k_ref[0], TOPK_WIDTH)
    hb = lax.bitcast_convert_type(h, jnp.int32)
    hi0 = jnp.max(hb, axis=1, keepdims=True)
    lo0 = jnp.full_like(hi0, -1)

    def body(_, carry):
        lo, hi = carry
        mid = lo + (hi - lo) // 2
        cnt = jnp.sum((hb > mid).astype(jnp.int32), axis=1, keepdims=True)
        ge = cnt >= kk
        return jnp.where(ge, mid, lo), jnp.where(ge, hi, mid)

    lo, _ = lax.fori_loop(0, 31, body, (lo0, hi0))
    h_m = jnp.where(hb > lo, h, 0.0)
    h_ref[...] = h_m

    recon_ref[...] = lax.dot_general(
        h_m, wd_ref[...],
        dimension_numbers=(((1,), (1,)), ((), ())),
        preferred_element_type=jnp.float32,
    )


@jax.jit
def _run(x, kk, W_enc, b_enc, W_dec):
    grid = (N_TOK // BT,)
    return pl.pallas_call(
        _fused_body,
        grid=grid,
        in_specs=[
            pl.BlockSpec(memory_space=pltpu.SMEM),
            pl.BlockSpec((BT, D_IN), lambda i: (i, 0)),
            pl.BlockSpec((D_HID, D_IN), lambda i: (0, 0)),
            pl.BlockSpec((1, D_HID), lambda i: (0, 0)),
            pl.BlockSpec((D_IN, D_HID), lambda i: (0, 0)),
        ],
        out_specs=[
            pl.BlockSpec((BT, D_IN), lambda i: (i, 0)),
            pl.BlockSpec((BT, D_HID), lambda i: (i, 0)),
        ],
        out_shape=[
            jax.ShapeDtypeStruct((N_TOK, D_IN), jnp.float32),
            jax.ShapeDtypeStruct((N_TOK, D_HID), jnp.float32),
        ],
    )(kk, x, W_enc, b_enc.reshape(1, D_HID), W_dec)


def kernel(x, k, W_enc, b_enc, W_dec):
    kk = jnp.asarray(k, jnp.int32).reshape(1)
    recon, h = _run(x, kk, W_enc, b_enc, W_dec)
    return (recon, h)


# speedup vs baseline: 14.4737x; 1.1516x over previous
import jax
import jax.numpy as jnp
from jax import lax
from jax.experimental import pallas as pl
from jax.experimental.pallas import tpu as pltpu

N_TOK = 8192
D_IN = 1024
D_HID = 4096
TOPK_WIDTH = 64
BT = 256


def _count_gt(v16, m16):
    c = (v16 > m16).astype(jnp.int16)
    w = D_HID // 2
    while w >= 128:
        c = c[:, :w] + c[:, w:]
        w //= 2
    return jnp.sum(c.astype(jnp.int32), axis=1, keepdims=True)


def _bisect(v16, target, lo0, hi0, iters):
    def body(_, carry):
        lo, hi = carry
        mid = lo + (hi - lo) // 2
        ge = _count_gt(v16, mid.astype(jnp.int16)) >= target
        return jnp.where(ge, mid, lo), jnp.where(ge, hi, mid)

    lo, _ = lax.fori_loop(0, iters, body, (lo0, hi0))
    return lo


def _fused_body(k_ref, x_ref, we_ref, be_ref, wd_ref, recon_ref, h_ref):
    h_pre = lax.dot_general(
        x_ref[...], we_ref[...],
        dimension_numbers=(((1,), (1,)), ((), ())),
        preferred_element_type=jnp.float32,
    )
    h = jnp.maximum(h_pre + be_ref[...], 0.0)

    kk = jnp.minimum(k_ref[0], TOPK_WIDTH)
    hb = lax.bitcast_convert_type(h, jnp.int32)

    hi16 = (hb >> 16).astype(jnp.int16)
    ones = jnp.ones((BT, 1), jnp.int32)
    t_hi = _bisect(hi16, kk, -1 * ones, 0x7F80 * ones, 16)
    bkt = t_hi + 1
    bkt16 = bkt.astype(jnp.int16)
    c_above = _count_gt(hi16, bkt16)
    j = kk - c_above

    eq = hi16 == bkt16
    ls = jnp.where(eq, ((hb & 0xFFFF) - 0x8000).astype(jnp.int16),
                   jnp.int16(-0x8000))
    t_lo = _bisect(ls, j, -0x8001 * ones, 0x7FFF * ones, 17)
    mask = (hi16 > bkt16) | (eq & ((ls > t_lo.astype(jnp.int16)) |
                                   (t_lo == -0x8001)))
    h_m = jnp.where(mask, h, 0.0)
    h_ref[...] = h_m

    recon_ref[...] = lax.dot_general(
        h_m, wd_ref[...],
        dimension_numbers=(((1,), (1,)), ((), ())),
        preferred_element_type=jnp.float32,
    )


@jax.jit
def _run(x, kk, W_enc, b_enc, W_dec):
    grid = (N_TOK // BT,)
    return pl.pallas_call(
        _fused_body,
        grid=grid,
        in_specs=[
            pl.BlockSpec(memory_space=pltpu.SMEM),
            pl.BlockSpec((BT, D_IN), lambda i: (i, 0)),
            pl.BlockSpec((D_HID, D_IN), lambda i: (0, 0)),
            pl.BlockSpec((1, D_HID), lambda i: (0, 0)),
            pl.BlockSpec((D_IN, D_HID), lambda i: (0, 0)),
        ],
        out_specs=[
            pl.BlockSpec((BT, D_IN), lambda i: (i, 0)),
            pl.BlockSpec((BT, D_HID), lambda i: (i, 0)),
        ],
        out_shape=[
            jax.ShapeDtypeStruct((N_TOK, D_IN), jnp.float32),
            jax.ShapeDtypeStruct((N_TOK, D_HID), jnp.float32),
        ],
    )(kk, x, W_enc, b_enc.reshape(1, D_HID), W_dec)


def kernel(x, k, W_enc, b_enc, W_dec):
    kk = jnp.asarray(k, jnp.int32).reshape(1)
    recon, h = _run(x, kk, W_enc, b_enc, W_dec)
    return (recon, h)


# speedup vs baseline: 15.1015x; 1.0434x over previous
import jax
import jax.numpy as jnp
from jax import lax
from jax.experimental import pallas as pl
from jax.experimental.pallas import tpu as pltpu

N_TOK = 8192
D_IN = 1024
D_HID = 4096
TOPK_WIDTH = 64
BT = 256
NSPLIT = 2


def _count_gt(v16, m16):
    c = (v16 > m16).astype(jnp.int16)
    w = D_HID // 2
    while w >= 128:
        c = c[:, :w] + c[:, w:]
        w //= 2
    return jnp.sum(c.astype(jnp.int32), axis=1, keepdims=True)


def _bisect_multi(vs, tgts, lo0s, hi0s, iters):
    def body(_, carry):
        los, his = carry
        mids = [lo + (hi - lo) // 2 for lo, hi in zip(los, his)]
        ges = [_count_gt(v, m.astype(jnp.int16)) >= t
               for v, m, t in zip(vs, mids, tgts)]
        los = tuple(jnp.where(g, m, lo) for g, m, lo in zip(ges, mids, los))
        his = tuple(jnp.where(g, hi, m) for g, m, hi in zip(ges, mids, his))
        return los, his

    los, _ = lax.fori_loop(0, iters, body, (tuple(lo0s), tuple(hi0s)))
    return los


def _topk_mask(h, kk):
    hb = lax.bitcast_convert_type(h, jnp.int32)
    rows = h.shape[0]
    sub = rows // NSPLIT
    hbs = [hb[i * sub:(i + 1) * sub] for i in range(NSPLIT)]
    ones = jnp.ones((sub, 1), jnp.int32)

    hi16s = [(b >> 16).astype(jnp.int16) for b in hbs]
    t_his = _bisect_multi(hi16s, [kk] * NSPLIT, [-1 * ones] * NSPLIT,
                          [0x7F80 * ones] * NSPLIT, 15)

    bkt16s = [(t + 1).astype(jnp.int16) for t in t_his]
    js = [kk - _count_gt(v, b) for v, b in zip(hi16s, bkt16s)]

    eqs = [v == b for v, b in zip(hi16s, bkt16s)]
    lss = [jnp.where(e, ((b & 0xFFFF) - 0x8000).astype(jnp.int16),
                     jnp.int16(-0x8000)) for e, b in zip(eqs, hbs)]
    t_los = _bisect_multi(lss, js, [-0x8001 * ones] * NSPLIT,
                          [0x7FFF * ones] * NSPLIT, 16)

    masks = [(v > b) | (e & ((ls > t.astype(jnp.int16)) | (t == -0x8001)))
             for v, b, e, ls, t in zip(hi16s, bkt16s, eqs, lss, t_los)]
    mask = jnp.concatenate(masks, axis=0)
    return jnp.where(mask, h, 0.0)


def _fused_body(k_ref, x_ref, we_ref, be_ref, wd_ref, recon_ref, h_ref):
    h_pre = lax.dot_general(
        x_ref[...], we_ref[...],
        dimension_numbers=(((1,), (1,)), ((), ())),
        preferred_element_type=jnp.float32,
    )
    h = jnp.maximum(h_pre + be_ref[...], 0.0)

    kk = jnp.minimum(k_ref[0], TOPK_WIDTH)
    h_m = _topk_mask(h, kk)
    h_ref[...] = h_m

    recon_ref[...] = lax.dot_general(
        h_m, wd_ref[...],
        dimension_numbers=(((1,), (1,)), ((), ())),
        preferred_element_type=jnp.float32,
    )


@jax.jit
def _run(x, kk, W_enc, b_enc, W_dec):
    grid = (N_TOK // BT,)
    return pl.pallas_call(
        _fused_body,
        grid=grid,
        in_specs=[
            pl.BlockSpec(memory_space=pltpu.SMEM),
            pl.BlockSpec((BT, D_IN), lambda i: (i, 0)),
            pl.BlockSpec((D_HID, D_IN), lambda i: (0, 0)),
            pl.BlockSpec((1, D_HID), lambda i: (0, 0)),
            pl.BlockSpec((D_IN, D_HID), lambda i: (0, 0)),
        ],
        out_specs=[
            pl.BlockSpec((BT, D_IN), lambda i: (i, 0)),
            pl.BlockSpec((BT, D_HID), lambda i: (i, 0)),
        ],
        out_shape=[
            jax.ShapeDtypeStruct((N_TOK, D_IN), jnp.float32),
            jax.ShapeDtypeStruct((N_TOK, D_HID), jnp.float32),
        ],
    )(kk, x, W_enc, b_enc.reshape(1, D_HID), W_dec)


def kernel(x, k, W_enc, b_enc, W_dec):
    kk = jnp.asarray(k, jnp.int32).reshape(1)
    recon, h = _run(x, kk, W_enc, b_enc, W_dec)
    return (recon, h)
